# baseline (speedup 1.0000x reference)
import functools

import jax
import jax.numpy as jnp
from jax import lax
from jax.experimental import pallas as pl
from jax.experimental.pallas import tpu as pltpu
from jax.experimental.pallas import tpu_sc as plsc


def _k1_body(ct_ref, sc_ref, idx_ref, w_ref):
    ct = ct_ref[0]
    s = sc_ref[0]
    R = ct.shape[1]
    S = s.shape[0]

    ab = jax.lax.dot_general(s, ct, (((1,), (0,)), ((), ())),
                             preferred_element_type=jnp.float32)
    cn = jnp.sum(ct * ct, axis=0, keepdims=True)
    sn = jnp.sum(s * s, axis=1, keepdims=True)
    sq = (-2.0 * ab + cn) + sn

    iota = jax.lax.broadcasted_iota(jnp.int32, (S, R), 0)
    d = sq
    firsts = []
    recips = []
    for _ in range(3):
        m = jnp.min(d, axis=0, keepdims=True)
        first = jnp.min(jnp.where(d == m, iota, S), axis=0,
                        keepdims=True)
        firsts.append(first)
        recips.append(1.0 / (m + 1e-8))
        d = jnp.where(iota == first, jnp.inf, d)
    norm = recips[0] + recips[1] + recips[2]
    gbase = pl.program_id(0) * S
    for k in range(3):
        idx_ref[k, 0, 0, :] = firsts[k][0] + gbase
        w_ref[k, 0, 0, :] = (recips[k] / norm)[0]


def _make_sc_interp(BN, D2, NC, NS, L, P, C):
    mesh = plsc.VectorSubcoreMesh(core_axis_name="c", subcore_axis_name="s")
    NCH = P // C
    NP = NCH // 2
    FB = D2 // L

    @functools.partial(
        pl.kernel, mesh=mesh,
        out_type=jax.ShapeDtypeStruct((BN, D2), jnp.float32),
        scratch_types=[
            pltpu.VMEM((3, P), jnp.int32),
            pltpu.VMEM((2, 3, C, D2), jnp.float32),
            pltpu.VMEM((2, 3, C, L), jnp.float32),
            pltpu.SemaphoreType.DMA,
            pltpu.SemaphoreType.DMA,
            pltpu.SemaphoreType.DMA,
            pltpu.SemaphoreType.DMA,
        ],
    )
    def sc_interp(idx_hbm, wr_hbm, table_hbm, out_hbm,
                  idx_v, rbuf, wbuf, g0, g1, s0, s1):
        wid = lax.axis_index("s") * NC + lax.axis_index("c")
        base = wid * P
        pltpu.sync_copy(idx_hbm.at[:, pl.ds(base, P)], idx_v)
        gsem = (g0, g1)
        ssem = (s0, s1)

        def fire_gather(ci, bi):
            co = ci * C
            for k in range(3):
                pltpu.async_copy(table_hbm.at[idx_v.at[k, pl.ds(co, C)]],
                                 rbuf.at[bi, k], gsem[bi])
            pltpu.async_copy(wr_hbm.at[:, pl.ds(base + co, C), :],
                             wbuf.at[bi], gsem[bi])

        def wait_gather(bi):
            for k in range(3):
                pltpu.make_async_copy(table_hbm.at[pl.ds(0, C)],
                                      rbuf.at[bi, k], gsem[bi]).wait()
            pltpu.make_async_copy(wr_hbm.at[:, pl.ds(0, C), :],
                                  wbuf.at[bi], gsem[bi]).wait()

        def fire_store(ci, bi):
            pltpu.async_copy(rbuf.at[bi, 0],
                             out_hbm.at[pl.ds(base + ci * C, C)], ssem[bi])

        def wait_store(bi):
            pltpu.make_async_copy(rbuf.at[bi, 0],
                                  out_hbm.at[pl.ds(base, C)], ssem[bi]).wait()

        def combine(bi):
            def pbody(p, carry):
                w0 = wbuf[bi, 0, p, :]
                w1 = wbuf[bi, 1, p, :]
                w2 = wbuf[bi, 2, p, :]
                for f in range(FB):
                    sl = pl.ds(f * L, L)
                    rbuf[bi, 0, p, sl] = (w0 * rbuf[bi, 0, p, sl]
                                          + w1 * rbuf[bi, 1, p, sl]
                                          + w2 * rbuf[bi, 2, p, sl])
                return carry

            lax.fori_loop(0, C, pbody, 0)

        fire_gather(0, 0)

        def pair(pi, carry):
            ciA = 2 * pi

            @pl.when(pi > 0)
            def _():
                wait_store(1)

            fire_gather(ciA + 1, 1)
            wait_gather(0)
            combine(0)
            fire_store(ciA, 0)

            @pl.when(pi < NP - 1)
            def _():
                wait_store(0)
                fire_gather(ciA + 2, 0)

            wait_gather(1)
            combine(1)
            fire_store(ciA + 1, 1)
            return carry

        lax.fori_loop(0, NP, pair, 0)
        wait_store(0)
        wait_store(1)

    return sc_interp


def _k2_body(data_ref, interp_ref, w0_ref, b0_ref, y1_ref, ssum_ref, ssq_ref):
    D1 = data_ref.shape[2]
    y = (jax.lax.dot_general(data_ref[0], w0_ref[:D1], (((1,), (0,)), ((), ())),
                             preferred_element_type=jnp.float32)
         + jax.lax.dot_general(interp_ref[0], w0_ref[D1:],
                               (((1,), (0,)), ((), ())),
                               preferred_element_type=jnp.float32)
         + b0_ref[:])
    y1_ref[0] = y.astype(jnp.bfloat16)
    _acc_stats(y, ssum_ref, ssq_ref)


def _acc_stats(y, ssum_ref, ssq_ref):
    bs = jnp.sum(y, axis=0, keepdims=True)
    bq = jnp.sum(y * y, axis=0, keepdims=True)
    first_step = (pl.program_id(0) == 0) & (pl.program_id(1) == 0)

    @pl.when(first_step)
    def _():
        ssum_ref[:] = bs
        ssq_ref[:] = bq

    @pl.when(jnp.logical_not(first_step))
    def _():
        ssum_ref[:] += bs
        ssq_ref[:] += bq


def _k3_body(y1_ref, sc_ref, sh_ref, w1_ref, b1_ref,
             y2_ref, ssum_ref, ssq_ref):
    x = jax.nn.relu(y1_ref[0].astype(jnp.float32) * sc_ref[:] + sh_ref[:])
    y = jax.lax.dot_general(x, w1_ref[:], (((1,), (0,)), ((), ())),
                            preferred_element_type=jnp.float32) + b1_ref[:]
    y2_ref[0] = y.astype(jnp.bfloat16)
    _acc_stats(y, ssum_ref, ssq_ref)


def _k4_body(y2_ref, sc_ref, sh_ref, out_ref):
    out_ref[0] = jax.nn.relu(y2_ref[0].astype(jnp.float32) * sc_ref[:]
                             + sh_ref[:])


def _scale_shift(ssum, ssq, g, beta, count):
    mean = ssum[0] / count
    var = ssq[0] / count - mean * mean
    a = g / jnp.sqrt(var + 1e-5)
    c = beta - mean * a
    return a.reshape(1, -1), c.reshape(1, -1)


@functools.partial(jax.jit, static_argnames=("row_block",))
def _forward_impl(coords, sample_coords, data, sample_data,
                  W0, b0, g0, beta0, W1, b1, g1, beta1, row_block=1024):
    B, N, _ = coords.shape
    S = sample_coords.shape[1]
    D1 = data.shape[2]
    D2 = sample_data.shape[2]
    H0 = W0.shape[1]
    H1 = W1.shape[1]
    R = row_block
    BN = B * N
    grid = (B, N // R)
    R2 = 1024
    grid2 = (B, N // R2)

    coords_t = coords.transpose(0, 2, 1)

    idx3, w3 = pl.pallas_call(
        _k1_body,
        grid=grid,
        in_specs=[
            pl.BlockSpec((1, 3, R), lambda b, i: (b, 0, i)),
            pl.BlockSpec((1, S, 3), lambda b, i: (b, 0, 0)),
        ],
        out_specs=[
            pl.BlockSpec((3, 1, 1, R), lambda b, i: (0, b, 0, i)),
            pl.BlockSpec((3, 1, 1, R), lambda b, i: (0, b, 0, i)),
        ],
        out_shape=[
            jax.ShapeDtypeStruct((3, B, 1, N), jnp.int32),
            jax.ShapeDtypeStruct((3, B, 1, N), jnp.float32),
        ],
    )(coords_t, sample_coords)

    info = plsc.get_sparse_core_info()
    NC, NS, L = info.num_cores, info.num_subcores, info.num_lanes
    P = BN // (NC * NS)
    sc_interp = _make_sc_interp(BN, D2, NC, NS, L, P, 32)
    wrep = jnp.broadcast_to(w3.reshape(3, BN, 1), (3, BN, L))
    interp = sc_interp(idx3.reshape(3, BN), wrep,
                       sample_data.reshape(B * S, D2))
    interp = interp.reshape(B, N, D2)

    y1, ssum1, ssq1 = pl.pallas_call(
        _k2_body,
        grid=grid2,
        in_specs=[
            pl.BlockSpec((1, R2, D1), lambda b, i: (b, i, 0)),
            pl.BlockSpec((1, R2, D2), lambda b, i: (b, i, 0)),
            pl.BlockSpec((D1 + D2, H0), lambda b, i: (0, 0)),
            pl.BlockSpec((1, H0), lambda b, i: (0, 0)),
        ],
        out_specs=[
            pl.BlockSpec((1, R2, H0), lambda b, i: (b, i, 0)),
            pl.BlockSpec((1, H0), lambda b, i: (0, 0)),
            pl.BlockSpec((1, H0), lambda b, i: (0, 0)),
        ],
        out_shape=[
            jax.ShapeDtypeStruct((B, N, H0), jnp.bfloat16),
            jax.ShapeDtypeStruct((1, H0), jnp.float32),
            jax.ShapeDtypeStruct((1, H0), jnp.float32),
        ],
    )(data.astype(jnp.bfloat16), interp, W0, b0.reshape(1, -1))

    a1, c1 = _scale_shift(ssum1, ssq1, g0, beta0, float(BN))

    y2, ssum2, ssq2 = pl.pallas_call(
        _k3_body,
        grid=grid2,
        in_specs=[
            pl.BlockSpec((1, R2, H0), lambda b, i: (b, i, 0)),
            pl.BlockSpec((1, H0), lambda b, i: (0, 0)),
            pl.BlockSpec((1, H0), lambda b, i: (0, 0)),
            pl.BlockSpec((H0, H1), lambda b, i: (0, 0)),
            pl.BlockSpec((1, H1), lambda b, i: (0, 0)),
        ],
        out_specs=[
            pl.BlockSpec((1, R2, H1), lambda b, i: (b, i, 0)),
            pl.BlockSpec((1, H1), lambda b, i: (0, 0)),
            pl.BlockSpec((1, H1), lambda b, i: (0, 0)),
        ],
        out_shape=[
            jax.ShapeDtypeStruct((B, N, H1), jnp.bfloat16),
            jax.ShapeDtypeStruct((1, H1), jnp.float32),
            jax.ShapeDtypeStruct((1, H1), jnp.float32),
        ],
    )(y1, a1, c1, W1, b1.reshape(1, -1))

    a2, c2 = _scale_shift(ssum2, ssq2, g1, beta1, float(BN))

    out = pl.pallas_call(
        _k4_body,
        grid=grid2,
        in_specs=[
            pl.BlockSpec((1, R2, H1), lambda b, i: (b, i, 0)),
            pl.BlockSpec((1, H1), lambda b, i: (0, 0)),
            pl.BlockSpec((1, H1), lambda b, i: (0, 0)),
        ],
        out_specs=pl.BlockSpec((1, R2, H1), lambda b, i: (b, i, 0)),
        out_shape=jax.ShapeDtypeStruct((B, N, H1), jnp.float32),
    )(y2, a2, c2)

    return out


def kernel(coords, sample_coords, data, sample_data,
           W0, b0, g0, beta0, W1, b1, g1, beta1):
    return _forward_impl(coords, sample_coords, data, sample_data,
                         W0, b0, g0, beta0, W1, b1, g1, beta1)


# speedup vs baseline: 1.0120x; 1.0120x over previous
import functools

import jax
import jax.numpy as jnp
from jax import lax
from jax.experimental import pallas as pl
from jax.experimental.pallas import tpu as pltpu
from jax.experimental.pallas import tpu_sc as plsc


def _k1_body(ct_ref, sc_ref, idx_ref, w_ref):
    ct = ct_ref[0]
    s = sc_ref[0]
    R = ct.shape[1]
    S = s.shape[0]

    ab = jax.lax.dot_general(s, ct, (((1,), (0,)), ((), ())),
                             preferred_element_type=jnp.float32)
    cn = jnp.sum(ct * ct, axis=0, keepdims=True)
    sn = jnp.sum(s * s, axis=1, keepdims=True)
    sq = (-2.0 * ab + cn) + sn

    iota = jax.lax.broadcasted_iota(jnp.int32, (S, R), 0)
    d = sq
    firsts = []
    recips = []
    for _ in range(3):
        m = jnp.min(d, axis=0, keepdims=True)
        first = jnp.min(jnp.where(d == m, iota, S), axis=0,
                        keepdims=True)
        firsts.append(first)
        recips.append(1.0 / (m + 1e-8))
        d = jnp.where(iota == first, jnp.inf, d)
    norm = recips[0] + recips[1] + recips[2]
    gbase = pl.program_id(0) * S
    for k in range(3):
        idx_ref[k, 0, 0, :] = firsts[k][0] + gbase
        w_ref[k, 0, 0, :] = (recips[k] / norm)[0]


def _make_sc_interp(BN, D2, NC, NS, L, P, C):
    mesh = plsc.VectorSubcoreMesh(core_axis_name="c", subcore_axis_name="s")
    NCH = P // C
    NP = NCH // 2
    FB = D2 // L

    @functools.partial(
        pl.kernel, mesh=mesh,
        out_type=jax.ShapeDtypeStruct((BN, D2), jnp.float32),
        scratch_types=[
            pltpu.VMEM((3, P), jnp.int32),
            pltpu.VMEM((2, 3, C, D2), jnp.float32),
            pltpu.VMEM((2, 3, C, L), jnp.float32),
            pltpu.SemaphoreType.DMA,
            pltpu.SemaphoreType.DMA,
            pltpu.SemaphoreType.DMA,
            pltpu.SemaphoreType.DMA,
        ],
    )
    def sc_interp(idx_hbm, wr_hbm, table_hbm, out_hbm,
                  idx_v, rbuf, wbuf, g0, g1, s0, s1):
        wid = lax.axis_index("s") * NC + lax.axis_index("c")
        base = wid * P
        pltpu.sync_copy(idx_hbm.at[:, pl.ds(base, P)], idx_v)
        gsem = (g0, g1)
        ssem = (s0, s1)

        def fire_gather(ci, bi):
            co = ci * C
            for k in range(3):
                pltpu.async_copy(table_hbm.at[idx_v.at[k, pl.ds(co, C)]],
                                 rbuf.at[bi, k], gsem[bi])
            pltpu.async_copy(wr_hbm.at[:, pl.ds(base + co, C), :],
                             wbuf.at[bi], gsem[bi])

        def wait_gather(bi):
            for k in range(3):
                pltpu.make_async_copy(table_hbm.at[pl.ds(0, C)],
                                      rbuf.at[bi, k], gsem[bi]).wait()
            pltpu.make_async_copy(wr_hbm.at[:, pl.ds(0, C), :],
                                  wbuf.at[bi], gsem[bi]).wait()

        def fire_store(ci, bi):
            pltpu.async_copy(rbuf.at[bi, 0],
                             out_hbm.at[pl.ds(base + ci * C, C)], ssem[bi])

        def wait_store(bi):
            pltpu.make_async_copy(rbuf.at[bi, 0],
                                  out_hbm.at[pl.ds(base, C)], ssem[bi]).wait()

        def combine(bi):
            def pbody(p, carry):
                w0 = wbuf[bi, 0, p, :]
                w1 = wbuf[bi, 1, p, :]
                w2 = wbuf[bi, 2, p, :]
                for f in range(FB):
                    sl = pl.ds(f * L, L)
                    rbuf[bi, 0, p, sl] = (w0 * rbuf[bi, 0, p, sl]
                                          + w1 * rbuf[bi, 1, p, sl]
                                          + w2 * rbuf[bi, 2, p, sl])
                return carry

            lax.fori_loop(0, C, pbody, 0)

        fire_gather(0, 0)

        def pair(pi, carry):
            ciA = 2 * pi

            @pl.when(pi > 0)
            def _():
                wait_store(1)

            fire_gather(ciA + 1, 1)
            wait_gather(0)
            combine(0)
            fire_store(ciA, 0)

            @pl.when(pi < NP - 1)
            def _():
                wait_store(0)
                fire_gather(ciA + 2, 0)

            wait_gather(1)
            combine(1)
            fire_store(ciA + 1, 1)
            return carry

        lax.fori_loop(0, NP, pair, 0)
        wait_store(0)
        wait_store(1)

    return sc_interp


def _k2_body(data_ref, interp_ref, w0_ref, b0_ref, y1_ref, ssum_ref, ssq_ref):
    D1 = data_ref.shape[2]
    y = (jax.lax.dot_general(data_ref[0], w0_ref[:D1], (((1,), (0,)), ((), ())),
                             preferred_element_type=jnp.float32)
         + jax.lax.dot_general(interp_ref[0], w0_ref[D1:],
                               (((1,), (0,)), ((), ())),
                               preferred_element_type=jnp.float32)
         + b0_ref[:])
    y1_ref[0] = y.astype(jnp.bfloat16)
    _acc_stats(y, ssum_ref, ssq_ref)


def _acc_stats(y, ssum_ref, ssq_ref):
    bs = jnp.sum(y, axis=0, keepdims=True)
    bq = jnp.sum(y * y, axis=0, keepdims=True)
    first_step = (pl.program_id(0) == 0) & (pl.program_id(1) == 0)

    @pl.when(first_step)
    def _():
        ssum_ref[:] = bs
        ssq_ref[:] = bq

    @pl.when(jnp.logical_not(first_step))
    def _():
        ssum_ref[:] += bs
        ssq_ref[:] += bq


def _k3_body(y1_ref, sc_ref, sh_ref, w1_ref, b1_ref,
             y2_ref, ssum_ref, ssq_ref):
    x = jax.nn.relu(y1_ref[0].astype(jnp.float32) * sc_ref[:] + sh_ref[:])
    y = jax.lax.dot_general(x, w1_ref[:], (((1,), (0,)), ((), ())),
                            preferred_element_type=jnp.float32) + b1_ref[:]
    y2_ref[0] = y.astype(jnp.bfloat16)
    _acc_stats(y, ssum_ref, ssq_ref)


def _k4_body(y2_ref, sc_ref, sh_ref, out_ref):
    out_ref[0] = jax.nn.relu(y2_ref[0].astype(jnp.float32) * sc_ref[:]
                             + sh_ref[:])


def _scale_shift(ssum, ssq, g, beta, count):
    mean = ssum[0] / count
    var = ssq[0] / count - mean * mean
    a = g / jnp.sqrt(var + 1e-5)
    c = beta - mean * a
    return a.reshape(1, -1), c.reshape(1, -1)


@functools.partial(jax.jit, static_argnames=("row_block",))
def _forward_impl(coords, sample_coords, data, sample_data,
                  W0, b0, g0, beta0, W1, b1, g1, beta1, row_block=1024):
    B, N, _ = coords.shape
    S = sample_coords.shape[1]
    D1 = data.shape[2]
    D2 = sample_data.shape[2]
    H0 = W0.shape[1]
    H1 = W1.shape[1]
    R = row_block
    BN = B * N
    grid = (B, N // R)
    R2 = 1024
    grid2 = (B, N // R2)

    coords_t = coords.transpose(0, 2, 1)

    idx3, w3 = pl.pallas_call(
        _k1_body,
        grid=grid,
        in_specs=[
            pl.BlockSpec((1, 3, R), lambda b, i: (b, 0, i)),
            pl.BlockSpec((1, S, 3), lambda b, i: (b, 0, 0)),
        ],
        out_specs=[
            pl.BlockSpec((3, 1, 1, R), lambda b, i: (0, b, 0, i)),
            pl.BlockSpec((3, 1, 1, R), lambda b, i: (0, b, 0, i)),
        ],
        out_shape=[
            jax.ShapeDtypeStruct((3, B, 1, N), jnp.int32),
            jax.ShapeDtypeStruct((3, B, 1, N), jnp.float32),
        ],
    )(coords_t, sample_coords)

    info = plsc.get_sparse_core_info()
    NC, NS, L = info.num_cores, info.num_subcores, info.num_lanes
    P = BN // (NC * NS)
    sc_interp = _make_sc_interp(BN, D2, NC, NS, L, P, 32)
    wrep = jnp.broadcast_to(w3.reshape(3, BN, 1), (3, BN, L))
    interp = sc_interp(idx3.reshape(3, BN), wrep,
                       sample_data.reshape(B * S, D2))
    interp = interp.reshape(B, N, D2)

    y1, ssum1, ssq1 = pl.pallas_call(
        _k2_body,
        grid=grid2,
        in_specs=[
            pl.BlockSpec((1, R2, D1), lambda b, i: (b, i, 0)),
            pl.BlockSpec((1, R2, D2), lambda b, i: (b, i, 0)),
            pl.BlockSpec((D1 + D2, H0), lambda b, i: (0, 0)),
            pl.BlockSpec((1, H0), lambda b, i: (0, 0)),
        ],
        out_specs=[
            pl.BlockSpec((1, R2, H0), lambda b, i: (b, i, 0)),
            pl.BlockSpec((1, H0), lambda b, i: (0, 0)),
            pl.BlockSpec((1, H0), lambda b, i: (0, 0)),
        ],
        out_shape=[
            jax.ShapeDtypeStruct((B, N, H0), jnp.bfloat16),
            jax.ShapeDtypeStruct((1, H0), jnp.float32),
            jax.ShapeDtypeStruct((1, H0), jnp.float32),
        ],
    )(data, interp, W0, b0.reshape(1, -1))

    a1, c1 = _scale_shift(ssum1, ssq1, g0, beta0, float(BN))

    y2, ssum2, ssq2 = pl.pallas_call(
        _k3_body,
        grid=grid2,
        in_specs=[
            pl.BlockSpec((1, R2, H0), lambda b, i: (b, i, 0)),
            pl.BlockSpec((1, H0), lambda b, i: (0, 0)),
            pl.BlockSpec((1, H0), lambda b, i: (0, 0)),
            pl.BlockSpec((H0, H1), lambda b, i: (0, 0)),
            pl.BlockSpec((1, H1), lambda b, i: (0, 0)),
        ],
        out_specs=[
            pl.BlockSpec((1, R2, H1), lambda b, i: (b, i, 0)),
            pl.BlockSpec((1, H1), lambda b, i: (0, 0)),
            pl.BlockSpec((1, H1), lambda b, i: (0, 0)),
        ],
        out_shape=[
            jax.ShapeDtypeStruct((B, N, H1), jnp.bfloat16),
            jax.ShapeDtypeStruct((1, H1), jnp.float32),
            jax.ShapeDtypeStruct((1, H1), jnp.float32),
        ],
    )(y1, a1, c1, W1, b1.reshape(1, -1))

    a2, c2 = _scale_shift(ssum2, ssq2, g1, beta1, float(BN))

    out = pl.pallas_call(
        _k4_body,
        grid=grid2,
        in_specs=[
            pl.BlockSpec((1, R2, H1), lambda b, i: (b, i, 0)),
            pl.BlockSpec((1, H1), lambda b, i: (0, 0)),
            pl.BlockSpec((1, H1), lambda b, i: (0, 0)),
        ],
        out_specs=pl.BlockSpec((1, R2, H1), lambda b, i: (b, i, 0)),
        out_shape=jax.ShapeDtypeStruct((B, N, H1), jnp.float32),
    )(y2, a2, c2)

    return out


def kernel(coords, sample_coords, data, sample_data,
           W0, b0, g0, beta0, W1, b1, g1, beta1):
    return _forward_impl(coords, sample_coords, data, sample_data,
                         W0, b0, g0, beta0, W1, b1, g1, beta1)
